# baseline (speedup 1.0000x reference)
import functools

import jax
import jax.numpy as jnp
from jax import lax
from jax.experimental import pallas as pl
from jax.experimental.pallas import tpu as pltpu
from jax.experimental.pallas import tpu_sc as plsc

VOCAB = 119
FEAT = 200
D_MODEL = 512
VPAD = 128

NC, NS = 2, 16
NW = NC * NS
ROWS = 16384 * 20
RPW = ROWS // NW
CHUNK = 128
NCHUNK = RPW // CHUNK


def _proj_body(cbfv_ref, w_ref, b_ref, out_ref):
    out_ref[...] = (
        jnp.dot(cbfv_ref[...], w_ref[...], preferred_element_type=jnp.float32)
        + b_ref[...]
    )


def _compute_proj(cbfv_pad, W, b_row):
    return pl.pallas_call(
        _proj_body,
        out_shape=jax.ShapeDtypeStruct((VPAD, D_MODEL), jnp.float32),
    )(cbfv_pad, W, b_row)


_SC_MESH = plsc.VectorSubcoreMesh(
    core_axis_name="c", subcore_axis_name="s", num_cores=NC, num_subcores=NS
)


@functools.partial(
    pl.kernel,
    out_type=jax.ShapeDtypeStruct((ROWS, D_MODEL), jnp.float32),
    mesh=_SC_MESH,
    scratch_types=[
        pltpu.VMEM((RPW,), jnp.int32),
        pltpu.VMEM((CHUNK, D_MODEL), jnp.float32),
        pltpu.SemaphoreType.DMA,
    ],
)
def _sc_gather(idx_hbm, tab_hbm, out_hbm, idx_v, rows_v, sem):
    wid = lax.axis_index("s") * NC + lax.axis_index("c")
    base = wid * RPW
    pltpu.sync_copy(idx_hbm.at[pl.ds(base, RPW)], idx_v)

    def chunk_step(i, carry):
        row0 = i * CHUNK
        pltpu.async_copy(
            tab_hbm.at[idx_v.at[pl.ds(row0, CHUNK)]], rows_v, sem
        ).wait()
        pltpu.sync_copy(rows_v, out_hbm.at[pl.ds(base + row0, CHUNK)])
        return carry

    lax.fori_loop(0, NCHUNK, chunk_step, 0)


def kernel(src, cbfv, W, b):
    cbfv_pad = jnp.pad(cbfv, ((0, VPAD - VOCAB), (0, 0)))
    proj = _compute_proj(cbfv_pad, W, b.reshape(1, D_MODEL))
    idx = src.reshape(-1).astype(jnp.int32)
    out = _sc_gather(idx, proj)
    return out.reshape(src.shape[0], src.shape[1], D_MODEL)


# speedup vs baseline: 1.0033x; 1.0033x over previous
import functools

import jax
import jax.numpy as jnp
from jax import lax
from jax.experimental import pallas as pl
from jax.experimental.pallas import tpu as pltpu
from jax.experimental.pallas import tpu_sc as plsc

VOCAB = 119
FEAT = 200
D_MODEL = 512
VPAD = 128

NC, NS = 2, 16
NW = NC * NS
ROWS = 16384 * 20
RPW = ROWS // NW
CHUNK = 80
NCHUNK = RPW // CHUNK


def _proj_body(cbfv_ref, w_ref, b_ref, out_ref):
    out_ref[...] = (
        jnp.dot(cbfv_ref[...], w_ref[...], preferred_element_type=jnp.float32)
        + b_ref[...]
    )


def _compute_proj(cbfv_pad, W, b_row):
    return pl.pallas_call(
        _proj_body,
        out_shape=jax.ShapeDtypeStruct((VPAD, D_MODEL), jnp.float32),
    )(cbfv_pad, W, b_row)


_SC_MESH = plsc.VectorSubcoreMesh(
    core_axis_name="c", subcore_axis_name="s", num_cores=NC, num_subcores=NS
)


@functools.partial(
    pl.kernel,
    out_type=jax.ShapeDtypeStruct((ROWS, D_MODEL), jnp.float32),
    mesh=_SC_MESH,
    scratch_types=[
        pltpu.VMEM((RPW,), jnp.int32),
        pltpu.VMEM((CHUNK, D_MODEL), jnp.float32),
        pltpu.VMEM((CHUNK, D_MODEL), jnp.float32),
        pltpu.SemaphoreType.DMA,
        pltpu.SemaphoreType.DMA,
        pltpu.SemaphoreType.DMA,
        pltpu.SemaphoreType.DMA,
    ],
)
def _sc_gather(idx_hbm, tab_hbm, out_hbm, idx_v, rows0, rows1, g0, g1, w0, w1):
    wid = lax.axis_index("s") * NC + lax.axis_index("c")
    base = wid * RPW
    pltpu.sync_copy(idx_hbm.at[pl.ds(base, RPW)], idx_v)

    bufs = (rows0, rows1)
    gsem = (g0, g1)
    wsem = (w0, w1)

    def gather_start(i, b):
        pltpu.async_copy(
            tab_hbm.at[idx_v.at[pl.ds(i * CHUNK, CHUNK)]], bufs[b], gsem[b]
        )

    def gather_wait(i, b):
        pltpu.make_async_copy(
            tab_hbm.at[idx_v.at[pl.ds(i * CHUNK, CHUNK)]], bufs[b], gsem[b]
        ).wait()

    def write_start(i, b):
        pltpu.async_copy(bufs[b], out_hbm.at[pl.ds(base + i * CHUNK, CHUNK)], wsem[b])

    def write_wait(i, b):
        pltpu.make_async_copy(
            bufs[b], out_hbm.at[pl.ds(base + i * CHUNK, CHUNK)], wsem[b]
        ).wait()

    gather_start(0, 0)
    gather_wait(0, 0)
    write_start(0, 0)
    gather_start(1, 1)

    def pair_step(k, carry):
        i1 = 1 + 2 * k
        gather_wait(i1, 1)
        write_start(i1, 1)
        write_wait(i1 - 1, 0)
        gather_start(i1 + 1, 0)
        i2 = i1 + 1
        gather_wait(i2, 0)
        write_start(i2, 0)
        write_wait(i2 - 1, 1)
        gather_start(i2 + 1, 1)
        return carry

    lax.fori_loop(0, (NCHUNK - 2) // 2, pair_step, 0)

    last = NCHUNK - 1
    gather_wait(last, 1)
    write_start(last, 1)
    write_wait(last - 1, 0)
    write_wait(last, 1)


def kernel(src, cbfv, W, b):
    cbfv_pad = jnp.pad(cbfv, ((0, VPAD - VOCAB), (0, 0)))
    proj = _compute_proj(cbfv_pad, W, b.reshape(1, D_MODEL))
    idx = src.reshape(-1).astype(jnp.int32)
    out = _sc_gather(idx, proj)
    return out.reshape(src.shape[0], src.shape[1], D_MODEL)
